# Initial kernel scaffold; baseline (speedup 1.0000x reference)
import functools

import jax
import jax.numpy as jnp
from jax import lax
from jax.experimental import pallas as pl
from jax.experimental.pallas import tpu as pltpu
from jax.experimental.pallas import tpu_sc as plsc

_VOCAB = 1000000
_D = 32
_B = 4096
_L = 200

_INFO = plsc.get_sparse_core_info()
_NC = _INFO.num_cores
_NS = _INFO.num_subcores
_NW = _NC * _NS
_BPW = _B // _NW


def _sc_pool_body(ids_hbm, table_hbm, out_hbm, idx_v, acc_v, sem):
    c = lax.axis_index("c")
    s = lax.axis_index("s")
    wid = s * _NC + c
    base = wid * _BPW

    pltpu.sync_copy(ids_hbm.at[:, pl.ds(base, _BPW)], idx_v)

    def zbody(i, carry):
        zero = jnp.zeros((16,), jnp.float32)
        acc_v[i, pl.ds(0, 16)] = zero
        acc_v[i, pl.ds(16, 16)] = zero
        return carry

    lax.fori_loop(0, _BPW, zbody, 0)

    def gbody(j, carry):
        pltpu.async_copy(table_hbm.at[idx_v.at[j]], acc_v, sem, add=True)
        return carry

    lax.fori_loop(0, _L, gbody, 0)

    def wbody(j, carry):
        pltpu.make_async_copy(table_hbm.at[idx_v.at[0]], acc_v, sem).wait()
        return carry

    lax.fori_loop(0, _L, wbody, 0)

    pltpu.sync_copy(acc_v, out_hbm.at[pl.ds(base, _BPW), :])


@jax.jit
def _sc_pool(ids_t, table):
    mesh = plsc.VectorSubcoreMesh(core_axis_name="c", subcore_axis_name="s")
    f = pl.kernel(
        _sc_pool_body,
        out_type=jax.ShapeDtypeStruct((_B, _D), jnp.float32),
        mesh=mesh,
        scratch_types=[
            pltpu.VMEM((_L, _BPW), jnp.int32),
            pltpu.VMEM((_BPW, _D), jnp.float32),
            pltpu.SemaphoreType.DMA,
        ],
    )
    return f(ids_t, table)


def _tc_linear_body(x_ref, wt_ref, b_ref, o_ref):
    o_ref[...] = (
        jnp.dot(x_ref[...], wt_ref[...], preferred_element_type=jnp.float32)
        + b_ref[...]
    )


@jax.jit
def _tc_linear(sums, wt_scaled, b2d):
    return pl.pallas_call(
        _tc_linear_body,
        out_shape=jax.ShapeDtypeStruct((_B, 2), jnp.float32),
    )(sums, wt_scaled, b2d)


def kernel(input_ids, embedding, W, b):
    ids_t = input_ids.T.astype(jnp.int32)
    sums = _sc_pool(ids_t, embedding)
    wt_scaled = (W.T / jnp.float32(_L)).astype(jnp.float32)
    b2d = b.reshape(1, 2).astype(jnp.float32)
    return _tc_linear(sums, wt_scaled, b2d)


# speedup vs baseline: 2.4579x; 2.4579x over previous
import functools

import jax
import jax.numpy as jnp
from jax import lax
from jax.experimental import pallas as pl
from jax.experimental.pallas import tpu as pltpu
from jax.experimental.pallas import tpu_sc as plsc

_VOCAB = 1000000
_D = 32
_B = 4096
_L = 200

_INFO = plsc.get_sparse_core_info()
_NC = _INFO.num_cores
_NS = _INFO.num_subcores
_NW = _NC * _NS
_BPW = _B // _NW


def _sc_pool_body(ids_hbm, table_hbm, out_hbm, idx_v, acc_v, sem):
    c = lax.axis_index("c")
    s = lax.axis_index("s")
    wid = s * _NC + c
    base = wid * _BPW

    pltpu.sync_copy(ids_hbm.at[:, pl.ds(base, _BPW)], idx_v)

    def zbody(i, carry):
        zero = jnp.zeros((16,), jnp.float32)
        acc_v[i, pl.ds(0, 16)] = zero
        acc_v[i, pl.ds(16, 16)] = zero
        return carry

    lax.fori_loop(0, _BPW, zbody, 0)

    def gbody(j, carry):
        pltpu.async_copy(table_hbm.at[idx_v.at[j]], acc_v, sem, add=True)
        return carry

    lax.fori_loop(0, _L, gbody, 0)

    def wbody(j, carry):
        pltpu.make_async_copy(table_hbm.at[idx_v.at[0]], acc_v, sem).wait()
        return carry

    lax.fori_loop(0, _L, wbody, 0)

    pltpu.sync_copy(acc_v, out_hbm.at[pl.ds(base, _BPW), :])


@jax.jit
def _sc_pool(ids_t, table):
    mesh = plsc.VectorSubcoreMesh(core_axis_name="c", subcore_axis_name="s")
    f = pl.kernel(
        _sc_pool_body,
        out_type=jax.ShapeDtypeStruct((_B, _D), jnp.float32),
        mesh=mesh,
        scratch_types=[
            pltpu.VMEM((_L, _BPW), jnp.int32),
            pltpu.VMEM((_BPW, _D), jnp.float32),
            pltpu.SemaphoreType.DMA,
        ],
        compiler_params=pltpu.CompilerParams(use_tc_tiling_on_sc=False),
    )
    return f(ids_t, table)


def _tc_linear_body(x_ref, wt_ref, b_ref, o_ref):
    o_ref[...] = (
        jnp.dot(x_ref[...], wt_ref[...], preferred_element_type=jnp.float32)
        + b_ref[...]
    )


@jax.jit
def _tc_linear(sums, wt_scaled, b2d):
    return pl.pallas_call(
        _tc_linear_body,
        out_shape=jax.ShapeDtypeStruct((_B, 2), jnp.float32),
    )(sums, wt_scaled, b2d)


def kernel(input_ids, embedding, W, b):
    ids_t = input_ids.T.astype(jnp.int32)
    sums = _sc_pool(ids_t, embedding)
    wt_scaled = (W.T / jnp.float32(_L)).astype(jnp.float32)
    b2d = b.reshape(1, 2).astype(jnp.float32)
    return _tc_linear(sums, wt_scaled, b2d)
